# Initial kernel scaffold; baseline (speedup 1.0000x reference)
import jax
import jax.numpy as jnp
from jax.experimental import pallas as pl


def kernel(metric, token_position):
    raise NotImplementedError("write your pallas kernel here")


# speedup vs baseline: 3.0827x; 3.0827x over previous
import jax
import jax.numpy as jnp
from jax import lax
from jax.experimental import pallas as pl
from jax.experimental.pallas import tpu as pltpu
from jax.experimental.pallas import tpu_sc as plsc

R = 512
T = 2048
HALF = T // 2
TOUT = T - R
C = 1024
N = 4

_HI = jax.lax.Precision.HIGHEST


def _tc_decide(odn_ref, evn_ref, g_ref, n1_ref, mg_ref):
    b = pl.program_id(0)
    od_n = odn_ref[0]
    ev_n = evn_ref[0]

    scores = lax.dot_general(od_n, ev_n, (((1,), (1,)), ((), ())),
                             preferred_element_type=jnp.float32)
    col = lax.broadcasted_iota(jnp.int32, (HALF, HALF), 1)
    scores = jnp.where(col == 0, -jnp.inf, scores)

    nm_col = jnp.max(scores, axis=1, keepdims=True)
    cand = jnp.where(scores == nm_col, col, T)
    node1_col = jnp.min(cand, axis=1, keepdims=True)

    i_col = lax.broadcasted_iota(jnp.int32, (HALF, 1), 0)
    k_row = lax.broadcasted_iota(jnp.int32, (1, HALF), 1)
    ident = (i_col == k_row).astype(jnp.float32)
    nm_row = lax.dot_general(nm_col, ident, (((0,), (0,)), ((), ())),
                             precision=_HI, preferred_element_type=jnp.float32)

    before = (nm_row > nm_col) | ((nm_row == nm_col) & (col < i_col))
    rank_col = jnp.sum(before.astype(jnp.int32), axis=1, keepdims=True)
    merged_col = rank_col < R
    kept_col = ~merged_col

    ltri = (col <= i_col).astype(jnp.float32)
    kept_f = kept_col.astype(jnp.float32)
    c_col = lax.dot_general(ltri, kept_f, (((1,), (0,)), ((), ())),
                            precision=_HI,
                            preferred_element_type=jnp.float32).astype(jnp.int32)
    cm1_col = c_col - kept_col.astype(jnp.int32)

    row_even_col = jnp.where(i_col == 0, 0, i_col + cm1_col)
    row_odd_col = i_col + c_col

    base_e = b * HALF
    base_o = N * HALF + b * HALF
    p_row = lax.broadcasted_iota(jnp.int32, (1, TOUT), 1)
    ge = (row_even_col == p_row).astype(jnp.int32) * (i_col + base_e)
    go = ((row_odd_col == p_row) & kept_col).astype(jnp.int32) * (i_col + base_o)
    g_ref[0] = jnp.sum(ge + go, axis=0, keepdims=True)

    n1_ref[0] = node1_col
    mg_ref[0] = merged_col.astype(jnp.int32)


def _tc_augment(od_ref, ev_ref, n1_ref, mg_ref, evaug_ref):
    od = od_ref[0]
    ev = ev_ref[0]
    n1 = n1_ref[0]
    mg = mg_ref[0]

    k_row = lax.broadcasted_iota(jnp.int32, (1, HALF), 1)
    amat = ((mg != 0) & (n1 == k_row)).astype(jnp.float32)
    dstadd = lax.dot_general(amat, od, (((0,), (0,)), ((), ())),
                             precision=_HI, preferred_element_type=jnp.float32)
    evaug_ref[0] = ev + dstadd


def _tc_stage(od_n, ev_n, od, ev):
    g3, n13, mg3 = pl.pallas_call(
        _tc_decide,
        grid=(N,),
        in_specs=[
            pl.BlockSpec((1, HALF, C), lambda b: (b, 0, 0)),
            pl.BlockSpec((1, HALF, C), lambda b: (b, 0, 0)),
        ],
        out_specs=[
            pl.BlockSpec((1, 1, TOUT), lambda b: (b, 0, 0)),
            pl.BlockSpec((1, HALF, 1), lambda b: (b, 0, 0)),
            pl.BlockSpec((1, HALF, 1), lambda b: (b, 0, 0)),
        ],
        out_shape=[
            jax.ShapeDtypeStruct((N, 1, TOUT), jnp.int32),
            jax.ShapeDtypeStruct((N, HALF, 1), jnp.int32),
            jax.ShapeDtypeStruct((N, HALF, 1), jnp.int32),
        ],
    )(od_n, ev_n)
    ev_aug = pl.pallas_call(
        _tc_augment,
        grid=(N,),
        in_specs=[
            pl.BlockSpec((1, HALF, C), lambda b: (b, 0, 0)),
            pl.BlockSpec((1, HALF, C), lambda b: (b, 0, 0)),
            pl.BlockSpec((1, HALF, 1), lambda b: (b, 0, 0)),
            pl.BlockSpec((1, HALF, 1), lambda b: (b, 0, 0)),
        ],
        out_specs=[pl.BlockSpec((1, HALF, C), lambda b: (b, 0, 0))],
        out_shape=[jax.ShapeDtypeStruct((N, HALF, C), jnp.float32)],
    )(od, ev, n13, mg3)[0]
    return g3, ev_aug


NW = 32
ROWS_PER_W = N * TOUT // NW
CHUNK = 32


def _sc_body(tab_hbm, g_hbm, out_hbm, idx_v, rows_v, sem):
    c = lax.axis_index("c")
    s = lax.axis_index("s")
    w = c * 16 + s
    for h in range(ROWS_PER_W // CHUNK):
        off = w * ROWS_PER_W + h * CHUNK
        pltpu.sync_copy(g_hbm.at[pl.ds(off, CHUNK)], idx_v)
        pltpu.async_copy(tab_hbm.at[idx_v], rows_v, sem).wait()
        pltpu.sync_copy(rows_v, out_hbm.at[pl.ds(off, CHUNK)])


def _sc_stage(table, g_flat):
    mesh = plsc.VectorSubcoreMesh(core_axis_name="c", subcore_axis_name="s")
    run = pl.kernel(
        _sc_body,
        mesh=mesh,
        out_type=jax.ShapeDtypeStruct((N * TOUT, C), jnp.float32),
        scratch_types=[
            pltpu.VMEM((CHUNK,), jnp.int32),
            pltpu.VMEM((CHUNK, C), jnp.float32),
            pltpu.SemaphoreType.DMA,
        ],
    )
    return run(table, g_flat)


def kernel(metric, token_position):
    n, t, c = metric.shape
    blk = metric.reshape(n, t // 2, 2, c)
    od = blk[:, :, 1, :]
    ev = blk[:, :, 0, :]
    m = metric[:, 1:, :]
    m_n = m / jnp.linalg.norm(m, axis=-1, keepdims=True)
    od_n = m_n[:, ::2, :]
    ev_n = jnp.concatenate(
        [jnp.zeros((n, 1, c), jnp.float32), m_n[:, 1::2, :]], axis=1)
    g3, ev_aug = _tc_stage(od_n, ev_n, od, ev)
    table = jnp.concatenate(
        [ev_aug.reshape(n * (t // 2), c), od.reshape(n * (t // 2), c)], axis=0)
    out = _sc_stage(table, g3.reshape(-1))
    return out.reshape(n, TOUT, c)


# speedup vs baseline: 4.4869x; 1.4555x over previous
import jax
import jax.numpy as jnp
from jax import lax
from jax.experimental import pallas as pl
from jax.experimental.pallas import tpu as pltpu
from jax.experimental.pallas import tpu_sc as plsc

R = 512
T = 2048
HALF = T // 2
TOUT = T - R
C = 1024
N = 4

_HI = jax.lax.Precision.HIGHEST


def _tc_decide(odn_ref, evn_ref, g_ref, n1_ref, mg_ref):
    b = pl.program_id(0)
    od_n = odn_ref[0]
    ev_n = evn_ref[0]

    scores = lax.dot_general(od_n, ev_n, (((1,), (1,)), ((), ())),
                             preferred_element_type=jnp.float32)
    col = lax.broadcasted_iota(jnp.int32, (HALF, HALF), 1)
    scores = jnp.where(col == 0, -jnp.inf, scores)

    nm_col = jnp.max(scores, axis=1, keepdims=True)
    cand = jnp.where(scores == nm_col, col, T)
    node1_col = jnp.min(cand, axis=1, keepdims=True)

    i_col = lax.broadcasted_iota(jnp.int32, (HALF, 1), 0)
    k_row = lax.broadcasted_iota(jnp.int32, (1, HALF), 1)
    ident = (i_col == k_row).astype(jnp.float32)
    nm_row = lax.dot_general(nm_col, ident, (((0,), (0,)), ((), ())),
                             precision=_HI, preferred_element_type=jnp.float32)

    before = (nm_row > nm_col) | ((nm_row == nm_col) & (col < i_col))
    rank_col = jnp.sum(before.astype(jnp.int32), axis=1, keepdims=True)
    merged_col = rank_col < R
    kept_col = ~merged_col

    ltri = (col <= i_col).astype(jnp.float32)
    kept_f = kept_col.astype(jnp.float32)
    c_col = lax.dot_general(ltri, kept_f, (((1,), (0,)), ((), ())),
                            precision=_HI,
                            preferred_element_type=jnp.float32).astype(jnp.int32)
    cm1_col = c_col - kept_col.astype(jnp.int32)

    row_even_col = jnp.where(i_col == 0, 0, i_col + cm1_col)
    row_odd_col = i_col + c_col

    base_e = 2 * b * HALF
    base_o = (2 * b + 1) * HALF
    p_row = lax.broadcasted_iota(jnp.int32, (1, TOUT), 1)
    ge = (row_even_col == p_row).astype(jnp.int32) * (i_col + base_e)
    go = ((row_odd_col == p_row) & kept_col).astype(jnp.int32) * (i_col + base_o)
    g_ref[0] = jnp.sum(ge + go, axis=0, keepdims=True)

    n1_ref[0] = node1_col
    mg_ref[0] = merged_col.astype(jnp.int32)


def _tc_table(od_ref, ev_ref, n1_ref, mg_ref, tab_ref):
    j = pl.program_id(1)

    @pl.when(j == 0)
    def _():
        od = od_ref[0]
        ev = ev_ref[0]
        n1 = n1_ref[0]
        mg = mg_ref[0]
        k_row = lax.broadcasted_iota(jnp.int32, (1, HALF), 1)
        amat = ((mg != 0) & (n1 == k_row)).astype(jnp.float32)
        dstadd = lax.dot_general(amat, od, (((0,), (0,)), ((), ())),
                                 precision=_HI,
                                 preferred_element_type=jnp.float32)
        tab_ref[0] = ev + dstadd

    @pl.when(j == 1)
    def _():
        tab_ref[0] = od_ref[0]


def _tc_stage(od_n, ev_n, od, ev):
    g3, n13, mg3 = pl.pallas_call(
        _tc_decide,
        grid=(N,),
        in_specs=[
            pl.BlockSpec((1, HALF, C), lambda b: (b, 0, 0)),
            pl.BlockSpec((1, HALF, C), lambda b: (b, 0, 0)),
        ],
        out_specs=[
            pl.BlockSpec((1, 1, TOUT), lambda b: (b, 0, 0)),
            pl.BlockSpec((1, HALF, 1), lambda b: (b, 0, 0)),
            pl.BlockSpec((1, HALF, 1), lambda b: (b, 0, 0)),
        ],
        out_shape=[
            jax.ShapeDtypeStruct((N, 1, TOUT), jnp.int32),
            jax.ShapeDtypeStruct((N, HALF, 1), jnp.int32),
            jax.ShapeDtypeStruct((N, HALF, 1), jnp.int32),
        ],
    )(od_n, ev_n)
    table = pl.pallas_call(
        _tc_table,
        grid=(N, 2),
        in_specs=[
            pl.BlockSpec((1, HALF, C), lambda b, j: (b, 0, 0)),
            pl.BlockSpec((1, HALF, C), lambda b, j: (b, 0, 0)),
            pl.BlockSpec((1, HALF, 1), lambda b, j: (b, 0, 0)),
            pl.BlockSpec((1, HALF, 1), lambda b, j: (b, 0, 0)),
        ],
        out_specs=[pl.BlockSpec((1, HALF, C), lambda b, j: (2 * b + j, 0, 0))],
        out_shape=[jax.ShapeDtypeStruct((2 * N, HALF, C), jnp.float32)],
    )(od, ev, n13, mg3)[0]
    return g3, table


NW = 32
ROWS_PER_W = N * TOUT // NW
CHUNK = 32


def _sc_body(tab_hbm, g_hbm, out_hbm, idx_v, rows_v, sem):
    c = lax.axis_index("c")
    s = lax.axis_index("s")
    w = c * 16 + s
    for h in range(ROWS_PER_W // CHUNK):
        off = w * ROWS_PER_W + h * CHUNK
        pltpu.sync_copy(g_hbm.at[pl.ds(off, CHUNK)], idx_v)
        pltpu.async_copy(tab_hbm.at[idx_v], rows_v, sem).wait()
        pltpu.sync_copy(rows_v, out_hbm.at[pl.ds(off, CHUNK)])


def _sc_stage(table, g_flat):
    mesh = plsc.VectorSubcoreMesh(core_axis_name="c", subcore_axis_name="s")
    run = pl.kernel(
        _sc_body,
        mesh=mesh,
        out_type=jax.ShapeDtypeStruct((N * TOUT, C), jnp.float32),
        scratch_types=[
            pltpu.VMEM((CHUNK,), jnp.int32),
            pltpu.VMEM((CHUNK, C), jnp.float32),
            pltpu.SemaphoreType.DMA,
        ],
    )
    return run(table, g_flat)


def kernel(metric, token_position):
    n, t, c = metric.shape
    blk = metric.reshape(n, t // 2, 2, c)
    od = blk[:, :, 1, :]
    ev = blk[:, :, 0, :]
    od_n = od / jnp.linalg.norm(od, axis=-1, keepdims=True)
    ev_n = ev / jnp.linalg.norm(ev, axis=-1, keepdims=True)
    g3, table = _tc_stage(od_n, ev_n, od, ev)
    out = _sc_stage(table.reshape(2 * n * (t // 2), c), g3.reshape(-1))
    return out.reshape(n, TOUT, c)
